# baseline (speedup 1.0000x reference)
import functools

import jax
import jax.numpy as jnp
from jax import lax
from jax.experimental import pallas as pl
from jax.experimental.pallas import tpu as pltpu
from jax.experimental.pallas import tpu_sc as plsc

EMBED = 32
BATCH = 16384
NC, NS, L = 2, 16, 16
NW = NC * NS
BPW = BATCH // NW
CH = 128
NCH = BPW // CH

_mesh = plsc.VectorSubcoreMesh(core_axis_name="c", subcore_axis_name="s")


@functools.partial(
    pl.kernel,
    out_type=jax.ShapeDtypeStruct((BATCH,), jnp.float32),
    mesh=_mesh,
    compiler_params=pltpu.CompilerParams(
        use_tc_tiling_on_sc=False, needs_layout_passes=False),
    scratch_types=[
        pltpu.VMEM((NCH, CH), jnp.int32),
        pltpu.VMEM((NCH, CH), jnp.int32),
        pltpu.VMEM((BPW, EMBED), jnp.float32),
        pltpu.VMEM((BPW, EMBED), jnp.float32),
        pltpu.VMEM((BPW,), jnp.float32),
        pltpu.SemaphoreType.DMA,
    ],
)
def _sc_dots(x_hbm, y_hbm, cen_hbm, outw_hbm, dots_hbm, xv, yv, cbuf, obuf, dv,
             sem):
    wid = lax.axis_index("s") * NC + lax.axis_index("c")
    base = wid * BPW
    pltpu.sync_copy(x_hbm.at[pl.ds(wid * NCH, NCH)], xv)
    pltpu.sync_copy(y_hbm.at[pl.ds(wid * NCH, NCH)], yv)
    copies = []
    for j in range(NCH):
        copies.append(
            pltpu.async_copy(cen_hbm.at[xv.at[j]],
                             cbuf.at[pl.ds(j * CH, CH)], sem))
        copies.append(
            pltpu.async_copy(outw_hbm.at[yv.at[j]],
                             obuf.at[pl.ds(j * CH, CH)], sem))
    for cp in copies:
        cp.wait()

    lane = lax.iota(jnp.int32, L)

    def body(g, carry):
        rows = g * L + lane
        acc = jnp.zeros((L,), jnp.float32)
        for c in range(EMBED):
            col = jnp.full((L,), c, jnp.int32)
            a = plsc.load_gather(cbuf, [rows, col])
            b = plsc.load_gather(obuf, [rows, col])
            acc = acc + a * b
        dv[pl.ds(g * L, L)] = acc
        return carry

    lax.fori_loop(0, BPW // L, body, 0)
    pltpu.sync_copy(dv, dots_hbm.at[pl.ds(base, BPW)])


def _tc_loss_body(d_ref, o_ref):
    d = d_ref[...]
    neg_abs = -jnp.abs(d)
    ls = jnp.minimum(d, 0.0) - jnp.log(1.0 + jnp.exp(neg_abs))
    o_ref[0, 0] = -jnp.sum(ls) / BATCH


_tc_loss = pl.pallas_call(
    _tc_loss_body,
    out_shape=jax.ShapeDtypeStruct((1, 1), jnp.float32),
    out_specs=pl.BlockSpec(memory_space=pltpu.SMEM),
)


def kernel(x, y, center_weight, out_weight):
    x2 = x.reshape(NW * NCH, CH)
    y2 = y.reshape(NW * NCH, CH)
    dots = _sc_dots(x2, y2, center_weight, out_weight)
    loss = _tc_loss(dots.reshape(BATCH // 128, 128))
    return loss[0, 0]
